# baseline (speedup 1.0000x reference)
import jax
import jax.numpy as jnp
from jax.experimental import pallas as pl
from jax.experimental.pallas import tpu as pltpu

B, S, C, P = 16, 2048, 128, 256
TBLK = 128
NBLK = S // TBLK


def _fused_kernel(vecs_ref, w_hi_ref, w_lo_ref, eos_ref, b1_ref, wdec_ref,
                  post_hbm, nt_hbm, mask_hbm, eosp_hbm,
                  post_s, log_s, nt_s, mask_s, eosp_s, sems):
    i = pl.program_id(0)
    v = vecs_ref[...]
    d = (jnp.dot(v[:, :P // 2], w_hi_ref[...], preferred_element_type=jnp.float32) +
         jnp.dot(v[:, P // 2:], w_lo_ref[...], preferred_element_type=jnp.float32))
    d3 = d.reshape(B, TBLK, C)
    n2 = jnp.sum(d3 * d3, axis=-1)
    dt = jnp.sum(d3 * eos_ref[...][None], axis=-1)
    a = dt * jax.lax.rsqrt(n2)
    log_s[i] = jnp.where(a > 0, a, jnp.exp(a) - 1.0) + b1_ref[0, 0]
    r = jnp.dot(d3.reshape(B * TBLK, C), wdec_ref[...],
                preferred_element_type=jnp.float32)
    post_s[:, pl.ds(i * TBLK, TBLK), :] = r.reshape(B, TBLK, C)

    @pl.when(i == NBLK - 1)
    def _phase2():
        l = log_s[...]
        rm = jnp.max(jnp.max(l, axis=2, keepdims=True), axis=0, keepdims=True)
        se = jnp.sum(jnp.sum(jnp.exp(l - rm), axis=2, keepdims=True),
                     axis=0, keepdims=True)
        valid = jnp.logical_and(se < 2.0, rm > 0.0)
        gi = (jax.lax.broadcasted_iota(jnp.int32, l.shape, 0) * TBLK +
              jax.lax.broadcasted_iota(jnp.int32, l.shape, 2))
        idx = jnp.min(jnp.min(jnp.where(l == rm, gi, S), axis=2, keepdims=True),
                      axis=0, keepdims=True)
        nt = jnp.where(valid, idx, S)
        ntc = nt.reshape(B, 1)
        nt_s[...] = ntc
        iota_s = jax.lax.broadcasted_iota(jnp.int32, (B, S), 1)
        mask_s[...] = (iota_s > ntc).astype(jnp.int32)
        eosp_s[...] = (iota_s == ntc).astype(jnp.int32)
        small = [
            pltpu.make_async_copy(nt_s, nt_hbm, sems.at[B]),
            pltpu.make_async_copy(mask_s, mask_hbm, sems.at[B + 1]),
            pltpu.make_async_copy(eosp_s, eosp_hbm, sems.at[B + 2]),
        ]
        for cp in small:
            cp.start()
        gi2 = jax.lax.broadcasted_iota(jnp.int32, (S, C), 0)
        big = []
        for b in range(B):
            ntb = ntc[b:b + 1, :]
            post_s[b] = jnp.where(gi2 > ntb, 0.0, post_s[b])
            cp = pltpu.make_async_copy(post_s.at[b], post_hbm.at[b],
                                       sems.at[b])
            cp.start()
            big.append(cp)
        for cp in small:
            cp.wait()
        for cp in big:
            cp.wait()


@jax.jit
def kernel(vecs, W_decomp, W_dec, eos_vector, classifier1w, classifier1b):
    en = jnp.sqrt(jnp.sum(eos_vector * eos_vector))
    scale = jnp.abs(classifier1w[0]) / en
    eos_scaled = (eos_vector * scale).reshape(1, C)
    b1 = classifier1b.reshape(1, 1)

    post, nt, mask, eos_pos = pl.pallas_call(
        _fused_kernel,
        grid=(NBLK,),
        in_specs=[
            pl.BlockSpec((B, P), lambda i: (0, 0)),
            pl.BlockSpec((P // 2, TBLK * C), lambda i: (0, i)),
            pl.BlockSpec((P // 2, TBLK * C), lambda i: (1, i)),
            pl.BlockSpec((1, C), lambda i: (0, 0)),
            pl.BlockSpec((1, 1), lambda i: (0, 0), memory_space=pltpu.SMEM),
            pl.BlockSpec((C, C), lambda i: (0, 0)),
        ],
        out_specs=[
            pl.BlockSpec(memory_space=pl.ANY),
            pl.BlockSpec(memory_space=pl.ANY),
            pl.BlockSpec(memory_space=pl.ANY),
            pl.BlockSpec(memory_space=pl.ANY),
        ],
        out_shape=[
            jax.ShapeDtypeStruct((B, S, C), jnp.float32),
            jax.ShapeDtypeStruct((B, 1), jnp.int32),
            jax.ShapeDtypeStruct((B, S), jnp.int32),
            jax.ShapeDtypeStruct((B, S), jnp.int32),
        ],
        scratch_shapes=[
            pltpu.VMEM((B, S, C), jnp.float32),
            pltpu.VMEM((NBLK, B, TBLK), jnp.float32),
            pltpu.VMEM((B, 1), jnp.int32),
            pltpu.VMEM((B, S), jnp.int32),
            pltpu.VMEM((B, S), jnp.int32),
            pltpu.SemaphoreType.DMA((B + 3,)),
        ],
        compiler_params=pltpu.CompilerParams(
            dimension_semantics=("arbitrary",),
        ),
    )(vecs, W_decomp, W_decomp, eos_scaled, b1, W_dec)

    return (post, nt.reshape(B), mask, eos_pos)


# speedup vs baseline: 1.0164x; 1.0164x over previous
import jax
import jax.numpy as jnp
from jax.experimental import pallas as pl
from jax.experimental.pallas import tpu as pltpu

B, S, C, P = 16, 2048, 128, 256
TBLK = 128
NBLK = S // TBLK


def _fused_kernel(vecs_ref, w_ref, eos_ref, b1_ref, wdec_ref,
                  post_hbm, nt_hbm, mask_hbm, eosp_hbm,
                  post_s, log_s, nt_s, nt_smem, mask_s, eosp_s, sems):
    i = pl.program_id(0)
    d = jnp.dot(vecs_ref[...], w_ref[...], preferred_element_type=jnp.float32)
    d3 = d.reshape(B, TBLK, C)
    n2 = jnp.sum(d3 * d3, axis=-1)
    dt = jnp.sum(d3 * eos_ref[...][None], axis=-1)
    a = dt * jax.lax.rsqrt(n2)
    log_s[i] = jnp.where(a > 0, a, jnp.exp(a) - 1.0) + b1_ref[0, 0]
    r = jnp.dot(d3.reshape(B * TBLK, C), wdec_ref[...],
                preferred_element_type=jnp.float32)
    post_s[:, pl.ds(i * TBLK, TBLK), :] = r.reshape(B, TBLK, C)
    pltpu.make_async_copy(
        post_s.at[:, pl.ds(i * TBLK, TBLK), :],
        post_hbm.at[:, pl.ds(i * TBLK, TBLK), :],
        sems.at[i]).start()

    @pl.when(i == NBLK - 1)
    def _phase2():
        l = log_s[...]
        rm = jnp.max(jnp.max(l, axis=2, keepdims=True), axis=0, keepdims=True)
        se = jnp.sum(jnp.sum(jnp.exp(l - rm), axis=2, keepdims=True),
                     axis=0, keepdims=True)
        valid = jnp.logical_and(se < 2.0, rm > 0.0)
        gi = (jax.lax.broadcasted_iota(jnp.int32, l.shape, 0) * TBLK +
              jax.lax.broadcasted_iota(jnp.int32, l.shape, 2))
        idx = jnp.min(jnp.min(jnp.where(l == rm, gi, S), axis=2, keepdims=True),
                      axis=0, keepdims=True)
        nt = jnp.where(valid, idx, S)
        ntc = nt.reshape(B, 1)
        nt_s[...] = ntc
        iota_s = jax.lax.broadcasted_iota(jnp.int32, (B, S), 1)
        mask_s[...] = (iota_s > ntc).astype(jnp.int32)
        eosp_s[...] = (iota_s == ntc).astype(jnp.int32)
        small = [
            pltpu.make_async_copy(nt_s, nt_hbm, sems.at[NBLK]),
            pltpu.make_async_copy(mask_s, mask_hbm, sems.at[NBLK + 1]),
            pltpu.make_async_copy(eosp_s, eosp_hbm, sems.at[NBLK + 2]),
        ]
        for cp in small:
            cp.start()
        nt_to_smem = pltpu.make_async_copy(nt_s, nt_smem, sems.at[NBLK + 3])
        nt_to_smem.start()
        for j in range(NBLK):
            pltpu.make_async_copy(
                post_s.at[:, pl.ds(j * TBLK, TBLK), :],
                post_hbm.at[:, pl.ds(j * TBLK, TBLK), :],
                sems.at[j]).wait()
        nt_to_smem.wait()
        gi2 = jax.lax.broadcasted_iota(jnp.int32, (S, C), 0)
        for b in range(B):
            @pl.when(nt_smem[b, 0] < S - 1)
            def _fixup(b=b):
                ntb = ntc[b:b + 1, :]
                post_s[b] = jnp.where(gi2 > ntb, 0.0, post_s[b])
                cp = pltpu.make_async_copy(post_s.at[b], post_hbm.at[b],
                                           sems.at[NBLK + 3])
                cp.start()
                cp.wait()
        for cp in small:
            cp.wait()


@jax.jit
def kernel(vecs, W_decomp, W_dec, eos_vector, classifier1w, classifier1b):
    en = jnp.sqrt(jnp.sum(eos_vector * eos_vector))
    scale = jnp.abs(classifier1w[0]) / en
    eos_scaled = (eos_vector * scale).reshape(1, C)
    b1 = classifier1b.reshape(1, 1)

    post, nt, mask, eos_pos = pl.pallas_call(
        _fused_kernel,
        grid=(NBLK,),
        in_specs=[
            pl.BlockSpec((B, P), lambda i: (0, 0)),
            pl.BlockSpec((P, TBLK * C), lambda i: (0, i)),
            pl.BlockSpec((1, C), lambda i: (0, 0)),
            pl.BlockSpec((1, 1), lambda i: (0, 0), memory_space=pltpu.SMEM),
            pl.BlockSpec((C, C), lambda i: (0, 0)),
        ],
        out_specs=[
            pl.BlockSpec(memory_space=pl.ANY),
            pl.BlockSpec(memory_space=pl.ANY),
            pl.BlockSpec(memory_space=pl.ANY),
            pl.BlockSpec(memory_space=pl.ANY),
        ],
        out_shape=[
            jax.ShapeDtypeStruct((B, S, C), jnp.float32),
            jax.ShapeDtypeStruct((B, 1), jnp.int32),
            jax.ShapeDtypeStruct((B, S), jnp.int32),
            jax.ShapeDtypeStruct((B, S), jnp.int32),
        ],
        scratch_shapes=[
            pltpu.VMEM((B, S, C), jnp.float32),
            pltpu.VMEM((NBLK, B, TBLK), jnp.float32),
            pltpu.VMEM((B, 1), jnp.int32),
            pltpu.SMEM((B, 1), jnp.int32),
            pltpu.VMEM((B, S), jnp.int32),
            pltpu.VMEM((B, S), jnp.int32),
            pltpu.SemaphoreType.DMA((NBLK + 4,)),
        ],
        compiler_params=pltpu.CompilerParams(
            dimension_semantics=("arbitrary",),
            vmem_limit_bytes=64 * 1024 * 1024,
        ),
    )(vecs, W_decomp, eos_scaled, b1, W_dec)

    return (post, nt.reshape(B), mask, eos_pos)


# speedup vs baseline: 1.0187x; 1.0023x over previous
import jax
import jax.numpy as jnp
from jax.experimental import pallas as pl
from jax.experimental.pallas import tpu as pltpu

B, S, C, P = 16, 2048, 128, 256
TBLK = 64
NBLK = S // TBLK


def _fused_kernel(vecs_ref, w_ref, eos_ref, b1_ref, wdec_ref,
                  post_hbm, nt_hbm, mask_hbm, eosp_hbm,
                  post_s, log_s, nt_s, nt_smem, mask_s, eosp_s, sems):
    i = pl.program_id(0)
    d = jnp.dot(vecs_ref[...], w_ref[...], preferred_element_type=jnp.float32)
    d3 = d.reshape(B, TBLK, C)
    n2 = jnp.sum(d3 * d3, axis=-1)
    dt = jnp.sum(d3 * eos_ref[...][None], axis=-1)
    a = dt * jax.lax.rsqrt(n2)
    log_s[i] = jnp.where(a > 0, a, jnp.exp(a) - 1.0) + b1_ref[0, 0]
    r = jnp.dot(d3.reshape(B * TBLK, C), wdec_ref[...],
                preferred_element_type=jnp.float32)
    post_s[:, pl.ds(i * TBLK, TBLK), :] = r.reshape(B, TBLK, C)
    pltpu.make_async_copy(
        post_s.at[:, pl.ds(i * TBLK, TBLK), :],
        post_hbm.at[:, pl.ds(i * TBLK, TBLK), :],
        sems.at[i]).start()

    @pl.when(i == NBLK - 1)
    def _phase2():
        l = log_s[...]
        rm = jnp.max(jnp.max(l, axis=2, keepdims=True), axis=0, keepdims=True)
        se = jnp.sum(jnp.sum(jnp.exp(l - rm), axis=2, keepdims=True),
                     axis=0, keepdims=True)
        valid = jnp.logical_and(se < 2.0, rm > 0.0)
        gi = (jax.lax.broadcasted_iota(jnp.int32, l.shape, 0) * TBLK +
              jax.lax.broadcasted_iota(jnp.int32, l.shape, 2))
        idx = jnp.min(jnp.min(jnp.where(l == rm, gi, S), axis=2, keepdims=True),
                      axis=0, keepdims=True)
        nt = jnp.where(valid, idx, S)
        ntc = nt.reshape(B, 1)
        nt_s[...] = ntc
        iota_s = jax.lax.broadcasted_iota(jnp.int32, (B, S), 1)
        mask_s[...] = (iota_s > ntc).astype(jnp.int32)
        eosp_s[...] = (iota_s == ntc).astype(jnp.int32)
        small = [
            pltpu.make_async_copy(nt_s, nt_hbm, sems.at[NBLK]),
            pltpu.make_async_copy(mask_s, mask_hbm, sems.at[NBLK + 1]),
            pltpu.make_async_copy(eosp_s, eosp_hbm, sems.at[NBLK + 2]),
        ]
        for cp in small:
            cp.start()
        nt_to_smem = pltpu.make_async_copy(nt_s, nt_smem, sems.at[NBLK + 3])
        nt_to_smem.start()
        for j in range(NBLK):
            pltpu.make_async_copy(
                post_s.at[:, pl.ds(j * TBLK, TBLK), :],
                post_hbm.at[:, pl.ds(j * TBLK, TBLK), :],
                sems.at[j]).wait()
        nt_to_smem.wait()
        gi2 = jax.lax.broadcasted_iota(jnp.int32, (S, C), 0)
        for b in range(B):
            @pl.when(nt_smem[b, 0] < S - 1)
            def _fixup(b=b):
                ntb = ntc[b:b + 1, :]
                post_s[b] = jnp.where(gi2 > ntb, 0.0, post_s[b])
                cp = pltpu.make_async_copy(post_s.at[b], post_hbm.at[b],
                                           sems.at[NBLK + 3])
                cp.start()
                cp.wait()
        for cp in small:
            cp.wait()


@jax.jit
def kernel(vecs, W_decomp, W_dec, eos_vector, classifier1w, classifier1b):
    en = jnp.sqrt(jnp.sum(eos_vector * eos_vector))
    scale = jnp.abs(classifier1w[0]) / en
    eos_scaled = (eos_vector * scale).reshape(1, C)
    b1 = classifier1b.reshape(1, 1)

    post, nt, mask, eos_pos = pl.pallas_call(
        _fused_kernel,
        grid=(NBLK,),
        in_specs=[
            pl.BlockSpec((B, P), lambda i: (0, 0)),
            pl.BlockSpec((P, TBLK * C), lambda i: (0, i)),
            pl.BlockSpec((1, C), lambda i: (0, 0)),
            pl.BlockSpec((1, 1), lambda i: (0, 0), memory_space=pltpu.SMEM),
            pl.BlockSpec((C, C), lambda i: (0, 0)),
        ],
        out_specs=[
            pl.BlockSpec(memory_space=pl.ANY),
            pl.BlockSpec(memory_space=pl.ANY),
            pl.BlockSpec(memory_space=pl.ANY),
            pl.BlockSpec(memory_space=pl.ANY),
        ],
        out_shape=[
            jax.ShapeDtypeStruct((B, S, C), jnp.float32),
            jax.ShapeDtypeStruct((B, 1), jnp.int32),
            jax.ShapeDtypeStruct((B, S), jnp.int32),
            jax.ShapeDtypeStruct((B, S), jnp.int32),
        ],
        scratch_shapes=[
            pltpu.VMEM((B, S, C), jnp.float32),
            pltpu.VMEM((NBLK, B, TBLK), jnp.float32),
            pltpu.VMEM((B, 1), jnp.int32),
            pltpu.SMEM((B, 1), jnp.int32),
            pltpu.VMEM((B, S), jnp.int32),
            pltpu.VMEM((B, S), jnp.int32),
            pltpu.SemaphoreType.DMA((NBLK + 4,)),
        ],
        compiler_params=pltpu.CompilerParams(
            dimension_semantics=("arbitrary",),
            vmem_limit_bytes=64 * 1024 * 1024,
        ),
    )(vecs, W_decomp, eos_scaled, b1, W_dec)

    return (post, nt.reshape(B), mask, eos_pos)
